# baseline (speedup 1.0000x reference)
import functools

import jax
import jax.numpy as jnp
from jax import lax
from jax.experimental import pallas as pl
from jax.experimental.pallas import tpu as pltpu
from jax.experimental.pallas import tpu_sc as plsc

N_NODES = 10000
D = 128
NC = 2
NS = 16
NW = NC * NS
CHUNK = 128
NCH = 80
NE_PAD = NW * NCH * CHUNK
SLOW_CORE = 1
CH_FAST = 80
CH_SLOW = 80
PHASE = 40
ACC_ROWS = 10240
STRIPE = ACC_ROWS // NS
DEG_W = 128


def _make_seg_sum():
    mesh = plsc.VectorSubcoreMesh(core_axis_name="c", subcore_axis_name="s")
    scratch = [
        pltpu.VMEM((PHASE, CHUNK), jnp.int32),
        pltpu.VMEM((PHASE, CHUNK), jnp.int32),
        pltpu.VMEM((CHUNK, D), jnp.float32),
        pltpu.VMEM((CHUNK, D), jnp.float32),
        pltpu.VMEM_SHARED((ACC_ROWS, D), jnp.float32),
        pltpu.SemaphoreType.DMA,
        pltpu.SemaphoreType.DMA,
    ]

    def body(x_hbm, src_hbm, dst_hbm, out_hbm, src_v, dst_v, rows_a, rows_b,
             acc, sem_a, sem_b):
        c = lax.axis_index("c")
        s = lax.axis_index("s")

        zero16 = jnp.zeros((16,), jnp.float32)

        def zbody(i, _):
            rows_a[i // 8, pl.ds((i % 8) * 16, 16)] = zero16
            return 0

        lax.fori_loop(0, CHUNK * 8, zbody, 0)
        for jj in range(STRIPE // CHUNK):
            pltpu.sync_copy(rows_a, acc.at[pl.ds(s * STRIPE + jj * CHUNK, CHUNK)])

        plsc.subcore_barrier()

        count = jnp.where(c == SLOW_CORE, CH_SLOW, CH_FAST)
        base = jnp.where(c == SLOW_CORE, NS * CH_FAST + s * CH_SLOW, s * CH_FAST)

        def chunk_pair(jj, _):
            j0 = jj * 2
            pltpu.async_copy(x_hbm.at[src_v.at[j0 + 1]], rows_b, sem_b)
            pltpu.make_async_copy(x_hbm.at[src_v.at[j0]], rows_a, sem_a).wait()
            pltpu.sync_copy(rows_a, acc.at[dst_v.at[j0]], add=True)

            @pl.when(jj < PHASE // 2 - 1)
            def _():
                pltpu.async_copy(x_hbm.at[src_v.at[j0 + 2]], rows_a, sem_a)

            pltpu.make_async_copy(x_hbm.at[src_v.at[j0 + 1]], rows_b, sem_b).wait()
            pltpu.sync_copy(rows_b, acc.at[dst_v.at[j0 + 1]], add=True)
            return 0

        def phase_body(p, _):
            pltpu.sync_copy(src_hbm.at[pl.ds(base + p * PHASE, PHASE)], src_v)
            pltpu.sync_copy(dst_hbm.at[pl.ds(base + p * PHASE, PHASE)], dst_v)
            pltpu.async_copy(x_hbm.at[src_v.at[0]], rows_a, sem_a)
            lax.fori_loop(0, PHASE // 2, chunk_pair, 0)
            return 0

        lax.fori_loop(0, count // PHASE, phase_body, 0)

        plsc.subcore_barrier()
        pltpu.sync_copy(acc.at[pl.ds(s * STRIPE, STRIPE)],
                        out_hbm.at[c, pl.ds(s * STRIPE, STRIPE)])

    return pl.kernel(
        body, mesh=mesh,
        out_type=jax.ShapeDtypeStruct((NC, ACC_ROWS, D), jnp.float32),
        scratch_types=scratch,
    )


def _make_deg():
    mesh = plsc.VectorSubcoreMesh(core_axis_name="c", subcore_axis_name="s")
    scratch = [
        pltpu.VMEM((NCH, CHUNK), jnp.int32),
        pltpu.VMEM((CHUNK, DEG_W), jnp.float32),
        pltpu.VMEM_SHARED((ACC_ROWS, DEG_W), jnp.float32),
    ]

    def body(dst_hbm, deg_hbm, dst_v, ones_v, dacc):
        c = lax.axis_index("c")
        s = lax.axis_index("s")
        w = s * NC + c

        zero16 = jnp.zeros((16,), jnp.float32)

        def z16(i, _):
            ones_v[i // 8, pl.ds((i % 8) * 16, 16)] = zero16
            return 0

        lax.fori_loop(0, CHUNK * (DEG_W // 16), z16, 0)
        for jj in range(STRIPE // CHUNK):
            pltpu.sync_copy(ones_v, dacc.at[pl.ds(s * STRIPE + jj * CHUNK, CHUNK)])
        one16 = jnp.ones((16,), jnp.float32)

        def o16(i, _):
            ones_v[i // 8, pl.ds((i % 8) * 16, 16)] = one16
            return 0

        lax.fori_loop(0, CHUNK * (DEG_W // 16), o16, 0)

        plsc.subcore_barrier()
        pltpu.sync_copy(dst_hbm.at[pl.ds(w * NCH, NCH)], dst_v)

        def chunk_body(j, _):
            pltpu.sync_copy(ones_v, dacc.at[dst_v.at[j]], add=True)
            return 0

        lax.fori_loop(0, NCH, chunk_body, 0)

        plsc.subcore_barrier()
        pltpu.sync_copy(dacc.at[pl.ds(s * STRIPE, STRIPE)],
                        deg_hbm.at[c, pl.ds(s * STRIPE, STRIPE)])

    return pl.kernel(
        body, mesh=mesh,
        out_type=jax.ShapeDtypeStruct((NC, ACC_ROWS, DEG_W), jnp.float32),
        scratch_types=scratch,
    )


_seg_sum = _make_seg_sum()
_deg_count = _make_deg()

_BLK = 1000
_DEGC = 8


def _tc_a_body(x_ref, d0_ref, d1_ref, wa_ref, b_ref, o_ref):
    deg = d0_ref[:, 0:1] + d1_ref[:, 0:1]
    a = jnp.dot(x_ref[:, :], wa_ref[:, :], preferred_element_type=jnp.float32)
    o_ref[:, :] = deg * (a + b_ref[:, :])


_tc_a = pl.pallas_call(
    _tc_a_body,
    grid=(N_NODES // _BLK,),
    in_specs=[
        pl.BlockSpec((_BLK, D), lambda i: (i, 0)),
        pl.BlockSpec((_BLK, _DEGC), lambda i: (i, 0)),
        pl.BlockSpec((_BLK, _DEGC), lambda i: (i, 0)),
        pl.BlockSpec((D, D), lambda i: (0, 0)),
        pl.BlockSpec((1, D), lambda i: (0, 0)),
    ],
    out_specs=pl.BlockSpec((_BLK, D), lambda i: (i, 0)),
    out_shape=jax.ShapeDtypeStruct((N_NODES, D), jnp.float32),
)


def _tc_b_body(relu, a_ref, s0_ref, s1_ref, wb_ref, o_ref):
    t = a_ref[:, :] + jnp.dot(s0_ref[:, :] + s1_ref[:, :], wb_ref[:, :],
                              preferred_element_type=jnp.float32)
    o_ref[:, :] = jnp.maximum(t, 0.0) if relu else t


def _make_tc_b(relu):
    return pl.pallas_call(
        functools.partial(_tc_b_body, relu),
        grid=(N_NODES // _BLK,),
        in_specs=[
            pl.BlockSpec((_BLK, D), lambda i: (i, 0)),
            pl.BlockSpec((_BLK, D), lambda i: (i, 0)),
            pl.BlockSpec((_BLK, D), lambda i: (i, 0)),
            pl.BlockSpec((D, D), lambda i: (0, 0)),
        ],
        out_specs=pl.BlockSpec((_BLK, D), lambda i: (i, 0)),
        out_shape=jax.ShapeDtypeStruct((N_NODES, D), jnp.float32),
    )


_tc_b_relu = _make_tc_b(True)
_tc_b_lin = _make_tc_b(False)


def _layer(h, src2, dst2, d0, d1, W, b, tc_b):
    WT = W.T
    A = _tc_a(h, d0, d1, WT[:D], b.reshape(1, D))
    Sp = _seg_sum(h, src2, dst2)
    return tc_b(A, Sp[0, :N_NODES], Sp[1, :N_NODES], WT[D:])


def kernel(x, edge_index, W1, b1, W2, b2, W3, b3):
    src = edge_index[0].astype(jnp.int32)
    dst = edge_index[1].astype(jnp.int32)
    pad = NE_PAD - src.shape[0]
    ar = jnp.arange(pad, dtype=jnp.int32)
    src2 = jnp.concatenate([src, ar % N_NODES]).reshape(-1, CHUNK)
    dst2 = jnp.concatenate(
        [dst, N_NODES + ar % (ACC_ROWS - N_NODES)]).reshape(-1, CHUNK)
    Dp = _deg_count(dst2)
    d0 = Dp[0, :N_NODES, :_DEGC]
    d1 = Dp[1, :N_NODES, :_DEGC]
    h1 = _layer(x, src2, dst2, d0, d1, W1, b1, _tc_b_relu)
    h2 = _layer(h1, src2, dst2, d0, d1, W2, b2, _tc_b_relu)
    return _layer(h2, src2, dst2, d0, d1, W3, b3, _tc_b_lin)


# speedup vs baseline: 1.0013x; 1.0013x over previous
import functools

import jax
import jax.numpy as jnp
from jax import lax
from jax.experimental import pallas as pl
from jax.experimental.pallas import tpu as pltpu
from jax.experimental.pallas import tpu_sc as plsc

N_NODES = 10000
D = 128
NC = 2
NS = 16
NW = NC * NS
CHUNK = 128
NCH = 80
NE_PAD = NW * NCH * CHUNK
SLOW_CORE = 1
CH_FAST = 80
CH_SLOW = 80
PHASE = 40
ACC_ROWS = 10240
STRIPE = ACC_ROWS // NS
DEG_W = 128


def _make_seg_sum():
    mesh = plsc.VectorSubcoreMesh(core_axis_name="c", subcore_axis_name="s")
    scratch = [
        pltpu.VMEM((PHASE, CHUNK), jnp.int32),
        pltpu.VMEM((PHASE, CHUNK), jnp.int32),
        pltpu.VMEM((CHUNK, D), jnp.float32),
        pltpu.VMEM((CHUNK, D), jnp.float32),
        pltpu.VMEM_SHARED((ACC_ROWS, D), jnp.float32),
        pltpu.SemaphoreType.DMA,
        pltpu.SemaphoreType.DMA,
    ]

    def body(x_hbm, src_hbm, dst_hbm, out_hbm, src_v, dst_v, rows_a, rows_b,
             acc, sem_a, sem_b):
        c = lax.axis_index("c")
        s = lax.axis_index("s")

        zero16 = jnp.zeros((16,), jnp.float32)

        def zbody(i, _):
            rows_a[i // 8, pl.ds((i % 8) * 16, 16)] = zero16
            return 0

        lax.fori_loop(0, CHUNK * 8, zbody, 0)
        for jj in range(STRIPE // CHUNK):
            pltpu.sync_copy(rows_a, acc.at[pl.ds(s * STRIPE + jj * CHUNK, CHUNK)])

        plsc.subcore_barrier()

        count = jnp.where(c == SLOW_CORE, CH_SLOW, CH_FAST)
        base = jnp.where(c == SLOW_CORE, NS * CH_FAST + s * CH_SLOW, s * CH_FAST)

        def chunk_pair(jj, _):
            j0 = jj * 2
            pltpu.async_copy(x_hbm.at[src_v.at[j0 + 1]], rows_b, sem_b)
            pltpu.make_async_copy(x_hbm.at[src_v.at[j0]], rows_a, sem_a).wait()
            pltpu.sync_copy(rows_a, acc.at[dst_v.at[j0]], add=True)

            @pl.when(jj < PHASE // 2 - 1)
            def _():
                pltpu.async_copy(x_hbm.at[src_v.at[j0 + 2]], rows_a, sem_a)

            pltpu.make_async_copy(x_hbm.at[src_v.at[j0 + 1]], rows_b, sem_b).wait()
            pltpu.sync_copy(rows_b, acc.at[dst_v.at[j0 + 1]], add=True)
            return 0

        def phase_body(p, _):
            pltpu.sync_copy(src_hbm.at[pl.ds(base + p * PHASE, PHASE)], src_v)
            pltpu.sync_copy(dst_hbm.at[pl.ds(base + p * PHASE, PHASE)], dst_v)
            pltpu.async_copy(x_hbm.at[src_v.at[0]], rows_a, sem_a)
            lax.fori_loop(0, PHASE // 2, chunk_pair, 0)
            return 0

        lax.fori_loop(0, count // PHASE, phase_body, 0)

        plsc.subcore_barrier()
        pltpu.sync_copy(acc.at[pl.ds(s * STRIPE, STRIPE)],
                        out_hbm.at[c, pl.ds(s * STRIPE, STRIPE)])

    return pl.kernel(
        body, mesh=mesh,
        out_type=jax.ShapeDtypeStruct((NC, ACC_ROWS, D), jnp.float32),
        scratch_types=scratch,
    )


def _make_deg():
    mesh = plsc.VectorSubcoreMesh(core_axis_name="c", subcore_axis_name="s")
    scratch = [
        pltpu.VMEM((NCH, CHUNK), jnp.int32),
        pltpu.VMEM((CHUNK, DEG_W), jnp.float32),
        pltpu.VMEM_SHARED((ACC_ROWS, DEG_W), jnp.float32),
    ]

    def body(dst_hbm, deg_hbm, dst_v, ones_v, dacc):
        c = lax.axis_index("c")
        s = lax.axis_index("s")
        w = s * NC + c

        zero16 = jnp.zeros((16,), jnp.float32)

        def z16(i, _):
            ones_v[i // 8, pl.ds((i % 8) * 16, 16)] = zero16
            return 0

        lax.fori_loop(0, CHUNK * (DEG_W // 16), z16, 0)
        for jj in range(STRIPE // CHUNK):
            pltpu.sync_copy(ones_v, dacc.at[pl.ds(s * STRIPE + jj * CHUNK, CHUNK)])
        one16 = jnp.ones((16,), jnp.float32)

        def o16(i, _):
            ones_v[i // 8, pl.ds((i % 8) * 16, 16)] = one16
            return 0

        lax.fori_loop(0, CHUNK * (DEG_W // 16), o16, 0)

        plsc.subcore_barrier()
        pltpu.sync_copy(dst_hbm.at[pl.ds(w * NCH, NCH)], dst_v)

        def chunk_body(j, _):
            pltpu.sync_copy(ones_v, dacc.at[dst_v.at[j]], add=True)
            return 0

        lax.fori_loop(0, NCH, chunk_body, 0)

        plsc.subcore_barrier()
        pltpu.sync_copy(dacc.at[pl.ds(s * STRIPE, STRIPE)],
                        deg_hbm.at[c, pl.ds(s * STRIPE, STRIPE)])

    return pl.kernel(
        body, mesh=mesh,
        out_type=jax.ShapeDtypeStruct((NC, ACC_ROWS, DEG_W), jnp.float32),
        scratch_types=scratch,
    )


_seg_sum = _make_seg_sum()
_deg_count = _make_deg()

_BLK = 1000
_DEGC = 8


def _tc_body(relu, x_ref, s0_ref, s1_ref, d0_ref, d1_ref, wa_ref, wb_ref,
             b_ref, o_ref):
    deg = d0_ref[:, 0:1] + d1_ref[:, 0:1]
    a = jnp.dot(x_ref[:, :], wa_ref[:, :], preferred_element_type=jnp.float32)
    a = a + b_ref[:, :]
    t = deg * a + jnp.dot(s0_ref[:, :] + s1_ref[:, :], wb_ref[:, :],
                          preferred_element_type=jnp.float32)
    o_ref[:, :] = jnp.maximum(t, 0.0) if relu else t


def _make_tc(relu):
    return pl.pallas_call(
        functools.partial(_tc_body, relu),
        grid=(N_NODES // _BLK,),
        in_specs=[
            pl.BlockSpec((_BLK, D), lambda i: (i, 0)),
            pl.BlockSpec((_BLK, D), lambda i: (i, 0)),
            pl.BlockSpec((_BLK, D), lambda i: (i, 0)),
            pl.BlockSpec((_BLK, _DEGC), lambda i: (i, 0)),
            pl.BlockSpec((_BLK, _DEGC), lambda i: (i, 0)),
            pl.BlockSpec((D, D), lambda i: (0, 0)),
            pl.BlockSpec((D, D), lambda i: (0, 0)),
            pl.BlockSpec((1, D), lambda i: (0, 0)),
        ],
        out_specs=pl.BlockSpec((_BLK, D), lambda i: (i, 0)),
        out_shape=jax.ShapeDtypeStruct((N_NODES, D), jnp.float32),
    )


_tc_relu = _make_tc(True)
_tc_lin = _make_tc(False)


def _layer(tc, h, Sp, d0, d1, W, b):
    WT = W.T
    return tc(h, Sp[0, :N_NODES], Sp[1, :N_NODES], d0, d1,
              WT[:D], WT[D:], b.reshape(1, D))


def kernel(x, edge_index, W1, b1, W2, b2, W3, b3):
    src = edge_index[0].astype(jnp.int32)
    dst = edge_index[1].astype(jnp.int32)
    pad = NE_PAD - src.shape[0]
    ar = jnp.arange(pad, dtype=jnp.int32)
    src2 = jnp.concatenate([src, ar % N_NODES]).reshape(-1, CHUNK)
    dst2 = jnp.concatenate(
        [dst, N_NODES + ar % (ACC_ROWS - N_NODES)]).reshape(-1, CHUNK)
    Dp = _deg_count(dst2)
    d0 = Dp[0, :N_NODES, :_DEGC]
    d1 = Dp[1, :N_NODES, :_DEGC]
    S1p = _seg_sum(x, src2, dst2)
    h1 = _layer(_tc_relu, x, S1p, d0, d1, W1, b1)
    S2p = _seg_sum(h1, src2, dst2)
    h2 = _layer(_tc_relu, h1, S2p, d0, d1, W2, b2)
    S3p = _seg_sum(h2, src2, dst2)
    return _layer(_tc_lin, h2, S3p, d0, d1, W3, b3)
